# baseline (speedup 1.0000x reference)
import functools
import math

import jax
import jax.numpy as jnp
from jax import lax
from jax.experimental import pallas as pl
from jax.experimental.pallas import tpu as pltpu
from jax.experimental.pallas import tpu_sc as plsc

_SMOOTHING = 0.1
_PAD_IDX = 0

_VTC = 49920
_TC_VB = 3840
_VSC_END = 99968


def _sc_partials(pred2, tgt, eps, c_row):
    N, V = pred2.shape
    info = plsc.get_sparse_core_info()
    NW = info.num_cores * info.num_subcores
    NC = info.num_cores
    rpw = N // NW
    NT = (_VSC_END - _VTC) // 128
    CT = 32
    NFULL = (NT - 1) // CT
    CHUNKS = [CT] * NFULL + [NT - NFULL * CT]
    NCH = len(CHUNKS)
    CW = CT * 128

    mesh = plsc.VectorSubcoreMesh(core_axis_name="c", subcore_axis_name="s")

    @functools.partial(
        pl.kernel,
        mesh=mesh,
        out_type=jax.ShapeDtypeStruct((NW, 16), jnp.float32),
        scratch_types=[
            pltpu.VMEM((16,), jnp.int32),
            pltpu.VMEM((rpw, rpw, 128), jnp.float32),
            pltpu.VMEM((rpw, 128), jnp.float32),
            pltpu.VMEM((16,), jnp.float32),
            pltpu.VMEM((rpw, CW), jnp.float32),
            pltpu.VMEM((rpw, CW), jnp.float32),
            pltpu.VMEM((rpw, CW), jnp.float32),
            pltpu.SemaphoreType.DMA,
            pltpu.SemaphoreType.DMA,
            pltpu.SemaphoreType.DMA,
            pltpu.SemaphoreType.DMA,
        ],
    )
    def sc(pred_hbm, tgt_hbm, out_hbm, tgt_v, tval_v, pad_v, out_v,
           buf0, buf1, buf2, sem0, sem1, sem2, gsem):
        wid = lax.axis_index("s") * NC + lax.axis_index("c")
        base = pl.multiple_of(wid * rpw, rpw)
        pltpu.sync_copy(tgt_hbm.at[pl.ds(base, rpw)], tgt_v.at[pl.ds(0, rpw)])
        t = tgt_v[...]
        lane = lax.iota(jnp.int32, 16)

        gpad = pltpu.async_copy(
            pred_hbm.at[pl.ds(base, rpw), pl.ds(0, 128)], pad_v, gsem)
        gts = []
        for j in range(rpw):
            c0 = pl.multiple_of((t[j] // 128) * 128, 128)
            gts.append(pltpu.async_copy(
                pred_hbm.at[pl.ds(base, rpw), pl.ds(c0, 128)],
                tval_v.at[j], gsem))

        bufs = (buf0, buf1, buf2)
        sems = (sem0, sem1, sem2)
        ND = len(bufs)
        offs = [_VTC + sum(CHUNKS[:k]) * 128 for k in range(NCH)]

        def issue(k):
            w = CHUNKS[k] * 128
            src = pred_hbm.at[pl.ds(base, rpw), pl.ds(offs[k], w)]
            if CHUNKS[k] == CT:
                return pltpu.async_copy(src, bufs[k % ND], sems[k % ND])
            return pltpu.async_copy(src, bufs[k % ND].at[:, pl.ds(0, w)],
                                    sems[k % ND])

        handles = [None] * NCH
        for k in range(min(ND, NCH)):
            handles[k] = issue(k)

        zero = jnp.zeros((16,), jnp.float32)
        accs = (zero,) * rpw
        for k in range(NCH):
            handles[k].wait()
            buf = bufs[k % ND]
            nvr = CHUNKS[k] * 8

            @plsc.parallel_loop(0, nvr, 1, unroll=2, carry=accs)
            def red(i, a, buf=buf):
                return tuple(a[j] + buf[j, pl.ds(i * 16, 16)]
                             for j in range(rpw))

            accs = red
            if k + ND < NCH:
                handles[k + ND] = issue(k + ND)

        gpad.wait()
        for g in gts:
            g.wait()
        contrib = zero
        coef_t = eps - (1.0 - _SMOOTHING)
        for j in range(rpw):
            valid = t[j] != _PAD_IDX
            gsel = (t[j] % 128) // 16
            val = zero
            for g in range(8):
                val = val + jnp.where(gsel == g,
                                      tval_v[j, j, pl.ds(g * 16, 16)], zero)
            sparse = (jnp.where(lane == t[j] % 16, coef_t * val, 0.0)
                      + jnp.where(lane == 0,
                                  eps * pad_v[j, pl.ds(0, 16)] + c_row, 0.0))
            contrib = contrib + jnp.where(valid, sparse - eps * accs[j], zero)
        out_v[...] = contrib
        pltpu.sync_copy(out_v, out_hbm.at[wid])

    return sc(pred2, tgt)


def _tc_partial(p2, tail, t_col, eps):
    N = p2.shape[0]
    Vb = _TC_VB
    G = _VTC // Vb
    W = tail.shape[1]

    def body(t_ref, p_ref, tail_ref, out_ref):
        g = pl.program_id(0)
        x = p_ref[...]
        rows = jnp.dot(x, jnp.ones((Vb, 1), jnp.float32),
                       preferred_element_type=jnp.float32)
        m = t_ref[...] != _PAD_IDX
        val = -eps * jnp.sum(jnp.where(m, rows, 0.0))
        trows = jnp.dot(tail_ref[...], jnp.ones((W, 1), jnp.float32),
                        preferred_element_type=jnp.float32)
        tval = -eps * jnp.sum(jnp.where(m, trows, 0.0))
        val = val + jnp.where(g == 0, tval, 0.0)
        r0 = lax.broadcasted_iota(jnp.int32, (8, 128), 0) == 0
        l0 = lax.broadcasted_iota(jnp.int32, (8, 128), 1) == 0
        out_ref[...] = jnp.where(r0 & l0, val, 0.0)

    return pl.pallas_call(
        body,
        grid=(G,),
        in_specs=[
            pl.BlockSpec((N, 1), lambda g: (0, 0)),
            pl.BlockSpec((N, Vb), lambda g: (0, g)),
            pl.BlockSpec((N, W), lambda g: (0, 0)),
        ],
        out_specs=pl.BlockSpec((8, 128), lambda g: (g, 0)),
        out_shape=jax.ShapeDtypeStruct((8 * G, 128), jnp.float32),
        compiler_params=pltpu.CompilerParams(
            dimension_semantics=("parallel",)),
    )(t_col, p2, tail)


def _tc_combine(scp, tpart):
    def body(scp_ref, tp_ref, out_ref):
        out_ref[0, 0] = jnp.sum(scp_ref[...]) + jnp.sum(tp_ref[...])

    return pl.pallas_call(
        body,
        in_specs=[
            pl.BlockSpec(memory_space=pltpu.VMEM),
            pl.BlockSpec(memory_space=pltpu.VMEM),
        ],
        out_specs=pl.BlockSpec(memory_space=pltpu.SMEM),
        out_shape=jax.ShapeDtypeStruct((1, 1), jnp.float32),
    )(scp, tpart)


def kernel(pred, target):
    B, S, V = pred.shape
    N = B * S
    p2 = pred.reshape(N, V)
    t = target.reshape(N).astype(jnp.int32)
    eps = _SMOOTHING / (V - 2)
    c_row = (_SMOOTHING * math.log(eps)
             + (1.0 - _SMOOTHING) * math.log(1.0 - _SMOOTHING))
    scp = _sc_partials(p2, t, eps, c_row)
    tpart = _tc_partial(p2, p2[:, _VSC_END:], t.reshape(N, 1), eps)
    out = _tc_combine(scp, tpart)
    return out[0, 0]


# speedup vs baseline: 1.0290x; 1.0290x over previous
import functools
import math

import jax
import jax.numpy as jnp
from jax import lax
from jax.experimental import pallas as pl
from jax.experimental.pallas import tpu as pltpu
from jax.experimental.pallas import tpu_sc as plsc

_SMOOTHING = 0.1
_PAD_IDX = 0

_VTC = 53760
_TC_VB = 3840
_VSC_END = 99968


def _sc_partials(pred2, tgt, eps, c_row):
    N, V = pred2.shape
    info = plsc.get_sparse_core_info()
    NW = info.num_cores * info.num_subcores
    NC = info.num_cores
    rpw = N // NW
    NT = (_VSC_END - _VTC) // 128
    CT = 29
    NFULL = (NT - 1) // CT
    CHUNKS = [CT] * NFULL + [NT - NFULL * CT]
    NCH = len(CHUNKS)
    CW = CT * 128

    mesh = plsc.VectorSubcoreMesh(core_axis_name="c", subcore_axis_name="s")

    @functools.partial(
        pl.kernel,
        mesh=mesh,
        out_type=jax.ShapeDtypeStruct((NW, 16), jnp.float32),
        scratch_types=[
            pltpu.VMEM((16,), jnp.int32),
            pltpu.VMEM((rpw, rpw, 128), jnp.float32),
            pltpu.VMEM((rpw, 128), jnp.float32),
            pltpu.VMEM((16,), jnp.float32),
            pltpu.VMEM((rpw, CW), jnp.float32),
            pltpu.VMEM((rpw, CW), jnp.float32),
            pltpu.VMEM((rpw, CW), jnp.float32),
            pltpu.VMEM((rpw, CW), jnp.float32),
            pltpu.SemaphoreType.DMA,
            pltpu.SemaphoreType.DMA,
            pltpu.SemaphoreType.DMA,
            pltpu.SemaphoreType.DMA,
            pltpu.SemaphoreType.DMA,
        ],
    )
    def sc(pred_hbm, tgt_hbm, out_hbm, tgt_v, tval_v, pad_v, out_v,
           buf0, buf1, buf2, buf3, sem0, sem1, sem2, sem3, gsem):
        wid = lax.axis_index("s") * NC + lax.axis_index("c")
        base = pl.multiple_of(wid * rpw, rpw)
        pltpu.sync_copy(tgt_hbm.at[pl.ds(base, rpw)], tgt_v.at[pl.ds(0, rpw)])
        t = tgt_v[...]
        lane = lax.iota(jnp.int32, 16)

        gpad = pltpu.async_copy(
            pred_hbm.at[pl.ds(base, rpw), pl.ds(0, 128)], pad_v, gsem)
        gts = []
        for j in range(rpw):
            c0 = pl.multiple_of((t[j] // 128) * 128, 128)
            gts.append(pltpu.async_copy(
                pred_hbm.at[pl.ds(base, rpw), pl.ds(c0, 128)],
                tval_v.at[j], gsem))

        bufs = (buf0, buf1, buf2, buf3)
        sems = (sem0, sem1, sem2, sem3)
        ND = len(bufs)
        offs = [_VTC + sum(CHUNKS[:k]) * 128 for k in range(NCH)]

        def issue(k):
            w = CHUNKS[k] * 128
            src = pred_hbm.at[pl.ds(base, rpw), pl.ds(offs[k], w)]
            if CHUNKS[k] == CT:
                return pltpu.async_copy(src, bufs[k % ND], sems[k % ND])
            return pltpu.async_copy(src, bufs[k % ND].at[:, pl.ds(0, w)],
                                    sems[k % ND])

        handles = [None] * NCH
        for k in range(min(ND, NCH)):
            handles[k] = issue(k)

        zero = jnp.zeros((16,), jnp.float32)
        accs = (zero,) * rpw
        for k in range(NCH):
            handles[k].wait()
            buf = bufs[k % ND]
            nvr = CHUNKS[k] * 8

            @plsc.parallel_loop(0, nvr, 1, unroll=2, carry=accs)
            def red(i, a, buf=buf):
                return tuple(a[j] + buf[j, pl.ds(i * 16, 16)]
                             for j in range(rpw))

            accs = red
            if k + ND < NCH:
                handles[k + ND] = issue(k + ND)

        gpad.wait()
        for g in gts:
            g.wait()
        contrib = zero
        coef_t = eps - (1.0 - _SMOOTHING)
        for j in range(rpw):
            valid = t[j] != _PAD_IDX
            gsel = (t[j] % 128) // 16
            val = zero
            for g in range(8):
                val = val + jnp.where(gsel == g,
                                      tval_v[j, j, pl.ds(g * 16, 16)], zero)
            sparse = (jnp.where(lane == t[j] % 16, coef_t * val, 0.0)
                      + jnp.where(lane == 0,
                                  eps * pad_v[j, pl.ds(0, 16)] + c_row, 0.0))
            contrib = contrib + jnp.where(valid, sparse - eps * accs[j], zero)
        out_v[...] = contrib
        pltpu.sync_copy(out_v, out_hbm.at[wid])

    return sc(pred2, tgt)


def _tc_partial(p2, tail, t_col, eps):
    N = p2.shape[0]
    Vb = _TC_VB
    G = _VTC // Vb
    W = tail.shape[1]

    def body(t_ref, p_ref, tail_ref, out_ref):
        g = pl.program_id(0)
        x = p_ref[...]
        rows = jnp.dot(x, jnp.ones((Vb, 1), jnp.float32),
                       preferred_element_type=jnp.float32)
        m = t_ref[...] != _PAD_IDX
        val = -eps * jnp.sum(jnp.where(m, rows, 0.0))
        trows = jnp.dot(tail_ref[...], jnp.ones((W, 1), jnp.float32),
                        preferred_element_type=jnp.float32)
        tval = -eps * jnp.sum(jnp.where(m, trows, 0.0))
        val = val + jnp.where(g == 0, tval, 0.0)
        r0 = lax.broadcasted_iota(jnp.int32, (8, 128), 0) == 0
        l0 = lax.broadcasted_iota(jnp.int32, (8, 128), 1) == 0
        out_ref[...] = jnp.where(r0 & l0, val, 0.0)

    return pl.pallas_call(
        body,
        grid=(G,),
        in_specs=[
            pl.BlockSpec((N, 1), lambda g: (0, 0)),
            pl.BlockSpec((N, Vb), lambda g: (0, g)),
            pl.BlockSpec((N, W), lambda g: (0, 0)),
        ],
        out_specs=pl.BlockSpec((8, 128), lambda g: (g, 0)),
        out_shape=jax.ShapeDtypeStruct((8 * G, 128), jnp.float32),
        compiler_params=pltpu.CompilerParams(
            dimension_semantics=("parallel",)),
    )(t_col, p2, tail)


def _tc_combine(scp, tpart):
    def body(scp_ref, tp_ref, out_ref):
        out_ref[0, 0] = jnp.sum(scp_ref[...]) + jnp.sum(tp_ref[...])

    return pl.pallas_call(
        body,
        in_specs=[
            pl.BlockSpec(memory_space=pltpu.VMEM),
            pl.BlockSpec(memory_space=pltpu.VMEM),
        ],
        out_specs=pl.BlockSpec(memory_space=pltpu.SMEM),
        out_shape=jax.ShapeDtypeStruct((1, 1), jnp.float32),
    )(scp, tpart)


def kernel(pred, target):
    B, S, V = pred.shape
    N = B * S
    p2 = pred.reshape(N, V)
    t = target.reshape(N).astype(jnp.int32)
    eps = _SMOOTHING / (V - 2)
    c_row = (_SMOOTHING * math.log(eps)
             + (1.0 - _SMOOTHING) * math.log(1.0 - _SMOOTHING))
    scp = _sc_partials(p2, t, eps, c_row)
    tpart = _tc_partial(p2, p2[:, _VSC_END:], t.reshape(N, 1), eps)
    out = _tc_combine(scp, tpart)
    return out[0, 0]
